# baseline (speedup 1.0000x reference)
import functools

import jax
import jax.numpy as jnp
from jax import lax
from jax.experimental import pallas as pl
from jax.experimental.pallas import tpu as pltpu
from jax.experimental.pallas import tpu_sc as plsc

D = 300
L = 16
DR = 320
DH = DR // 2
B_SEG = 256
N_ROWS = 100000
NW = 32
R_BLK = 1024
N_BLK = (N_ROWS + R_BLK - 1) // R_BLK
N_PAD = N_BLK * R_BLK
CHUNK = 64
NC = 49
SEG_LEN = NC * CHUNK
ACC_W = B_SEG * DR
ACC_T = 82304


def _scale_block(xt_ref, wt_ref, b_ref, out_ref):
    xb = xt_ref[...]
    t = jax.lax.dot_general(wt_ref[...], xb, (((1,), (0,)), ((), ())),
                            preferred_element_type=jnp.float32)
    w = jax.nn.sigmoid(t + b_ref[0])
    s = jnp.transpose(xb * w, (1, 0))
    s = jnp.pad(s, ((0, 0), (0, DR - D)))
    be = jax.lax.bitcast_convert_type(s[:, :DH], jnp.uint32)
    bo = jax.lax.bitcast_convert_type(s[:, DH:], jnp.uint32)
    re = (be + 0x7FFF + ((be >> 16) & 1)) >> 16
    ro = (bo + 0x7FFF + ((bo >> 16) & 1)) & jnp.uint32(0xFFFF0000)
    out_ref[...] = jax.lax.bitcast_convert_type(re | ro, jnp.float32)


def _sc_pool_body(xw_hbm, seg_hbm, out_hbm, xbuf0, xbuf1, acc, segbuf,
                  sem0, sem1):
    cid = lax.axis_index("c")
    sid = lax.axis_index("s")
    wid = sid * 2 + cid
    rbase = ((wid * (N_ROWS // NW) + 7) // 8) * 8
    rend = (((wid + 1) * (N_ROWS // NW) + 7) // 8) * 8
    nrows = jnp.where(wid == NW - 1, N_ROWS - rbase, rend - rbase)

    pltpu.sync_copy(seg_hbm.at[pl.ds(rbase, SEG_LEN)],
                    segbuf.at[pl.ds(0, SEG_LEN)])

    zero = jnp.zeros((L,), jnp.float32)

    def zrow(i, carry):
        for u in range(8):
            acc[pl.ds((i * 8 + u) * L, L)] = zero
        return carry

    lax.fori_loop(0, ACC_T // (L * 8), zrow, 0)

    iota = lax.iota(jnp.int32, L)
    himask = jnp.full((L,), 0xFFFF0000, dtype=jnp.uint32)

    def xcopy(g, buf, sem):
        return pltpu.async_copy(xw_hbm.at[pl.ds(rbase + g * CHUNK, CHUNK), :],
                                buf, sem)

    def xwait(buf, sem):
        pltpu.make_async_copy(xw_hbm.at[pl.ds(0, CHUNK), :], buf, sem).wait()

    def process_row(buf, r, seg_scalar):
        soff = seg_scalar * DR
        for k in range(DH // L):
            v = buf[r, pl.ds(k * L, L)]
            u = plsc.bitcast(v, jnp.uint32)
            lo = plsc.bitcast(u << 16, jnp.float32)
            hi = plsc.bitcast(u & himask, jnp.float32)
            plsc.addupdate(acc.at[pl.ds(soff + k * L, L)], lo)
            plsc.addupdate(acc.at[pl.ds(soff + DH + k * L, L)], hi)

    def do_chunk(j, buf, sem, obuf, osem):
        xwait(buf, sem)

        @pl.when(j + 1 < NC)
        def _():
            xcopy(j + 1, obuf, osem)

        def grp(h, carry):
            lbase = j * CHUNK + h * L
            segv = segbuf[pl.ds(lbase, L)]
            segv = jnp.where(lbase + iota < nrows, segv, B_SEG)
            for u in range(L):
                process_row(buf, h * L + u, segv[u])
            return carry

        lax.fori_loop(0, CHUNK // L, grp, 0)

    xcopy(0, xbuf0, sem0)

    def pair(jp, carry):
        do_chunk(jp * 2, xbuf0, sem0, xbuf1, sem1)

        @pl.when(jp * 2 + 1 < NC)
        def _():
            do_chunk(jp * 2 + 1, xbuf1, sem1, xbuf0, sem0)

        return carry

    lax.fori_loop(0, (NC + 1) // 2, pair, 0)

    pltpu.sync_copy(acc.at[pl.ds(0, ACC_W)],
                    out_hbm.at[pl.ds(wid * ACC_W, ACC_W)])


_sc_pool = functools.partial(
    pl.kernel,
    out_type=jax.ShapeDtypeStruct((NW * ACC_W,), jnp.float32),
    mesh=plsc.VectorSubcoreMesh(core_axis_name="c", subcore_axis_name="s",
                                num_cores=2, num_subcores=16),
    compiler_params=pltpu.CompilerParams(use_tc_tiling_on_sc=True,
                                         needs_layout_passes=False),
    scratch_types=[
        pltpu.VMEM((CHUNK, DH), jnp.float32),
        pltpu.VMEM((CHUNK, DH), jnp.float32),
        pltpu.VMEM((ACC_T,), jnp.float32),
        pltpu.VMEM((SEG_LEN + L,), jnp.int32),
        pltpu.SemaphoreType.DMA,
        pltpu.SemaphoreType.DMA,
    ],
)(_sc_pool_body)


def _reduce_parts(p_ref, o_ref):
    o_ref[...] = jnp.sum(p_ref[...], axis=0)[:, :D]


def kernel(x, segment_ids, batch_size, W, b):
    del batch_size
    seg = jnp.pad(segment_ids.astype(jnp.int32), (0, SEG_LEN + L),
                  constant_values=B_SEG)
    xw = pl.pallas_call(
        _scale_block,
        grid=(N_BLK,),
        in_specs=[
            pl.BlockSpec((D, R_BLK), lambda i: (0, i)),
            pl.BlockSpec((1, D), lambda i: (0, 0)),
            pl.BlockSpec(memory_space=pltpu.SMEM),
        ],
        out_specs=pl.BlockSpec((R_BLK, DH), lambda i: (i, 0)),
        out_shape=jax.ShapeDtypeStruct((N_PAD, DH), jnp.float32),
    )(x.T, W.reshape(1, D), b.reshape(1))
    parts = _sc_pool(xw, seg)
    parts = parts.reshape(NW, B_SEG, DR)
    out = pl.pallas_call(
        _reduce_parts,
        out_shape=jax.ShapeDtypeStruct((B_SEG, D), jnp.float32),
    )(parts)
    return out


# speedup vs baseline: 1.3198x; 1.3198x over previous
import functools

import jax
import jax.numpy as jnp
from jax import lax
from jax.experimental import pallas as pl
from jax.experimental.pallas import tpu as pltpu
from jax.experimental.pallas import tpu_sc as plsc

D = 300
L = 16
DR = 320
DH = DR // 2
B_SEG = 256
N_ROWS = 100000
NW = 32
R_BLK = 1024
N_BLK = (N_ROWS + R_BLK - 1) // R_BLK
N_PAD = N_BLK * R_BLK
CHUNK = 64
NC = 49
SEG_LEN = NC * CHUNK
ACC_W = B_SEG * DR
ACC_T = 82304


def _scale_block(xt_ref, wt_ref, b_ref, out_ref):
    xb = xt_ref[...]
    t = jax.lax.dot_general(wt_ref[...], xb, (((1,), (0,)), ((), ())),
                            preferred_element_type=jnp.float32)
    w = jax.nn.sigmoid(t + b_ref[0])
    s = jnp.transpose(xb * w, (1, 0))
    s = jnp.pad(s, ((0, 0), (0, DR - D)))
    be = jax.lax.bitcast_convert_type(s[:, :DH], jnp.uint32)
    bo = jax.lax.bitcast_convert_type(s[:, DH:], jnp.uint32)
    re = (be + 0x7FFF + ((be >> 16) & 1)) >> 16
    ro = (bo + 0x7FFF + ((bo >> 16) & 1)) & jnp.uint32(0xFFFF0000)
    out_ref[...] = jax.lax.bitcast_convert_type(re | ro, jnp.float32)


def _sc_pool_body(xw_hbm, seg_hbm, out_hbm, xbuf0, xbuf1, acc, segbuf,
                  sem0, sem1):
    cid = lax.axis_index("c")
    sid = lax.axis_index("s")
    wid = sid * 2 + cid
    rbase = ((wid * (N_ROWS // NW) + 7) // 8) * 8
    rend = (((wid + 1) * (N_ROWS // NW) + 7) // 8) * 8
    nrows = jnp.where(wid == NW - 1, N_ROWS - rbase, rend - rbase)

    pltpu.sync_copy(seg_hbm.at[pl.ds(rbase, SEG_LEN)],
                    segbuf.at[pl.ds(0, SEG_LEN)])

    zero = jnp.zeros((L,), jnp.float32)

    def zrow(i, carry):
        for u in range(8):
            acc[pl.ds((i * 8 + u) * L, L)] = zero
        return carry

    lax.fori_loop(0, ACC_T // (L * 8), zrow, 0)

    iota = lax.iota(jnp.int32, L)
    himask = jnp.full((L,), 0xFFFF0000, dtype=jnp.uint32)

    def xcopy(g, buf, sem):
        return pltpu.async_copy(xw_hbm.at[pl.ds(rbase + g * CHUNK, CHUNK), :],
                                buf, sem)

    def xwait(buf, sem):
        pltpu.make_async_copy(xw_hbm.at[pl.ds(0, CHUNK), :], buf, sem).wait()

    def process_row(buf, r, seg_scalar):
        soff = seg_scalar * DR
        for k in range(DH // L):
            v = buf[r, pl.ds(k * L, L)]
            u = plsc.bitcast(v, jnp.uint32)
            lo = plsc.bitcast(u << 16, jnp.float32)
            hi = plsc.bitcast(u & himask, jnp.float32)
            plsc.addupdate(acc.at[pl.ds(soff + k * L, L)], lo)
            plsc.addupdate(acc.at[pl.ds(soff + DH + k * L, L)], hi)

    def do_chunk(j, buf, sem, obuf, osem):
        xwait(buf, sem)

        @pl.when(j + 1 < NC)
        def _():
            xcopy(j + 1, obuf, osem)

        def grp(h, carry):
            lbase = j * CHUNK + h * L
            segv = segbuf[pl.ds(lbase, L)]
            segv = jnp.where(lbase + iota < nrows, segv, B_SEG)
            seg0 = segv[0]
            uniform = jnp.all(segv == seg0)

            @pl.when(uniform)
            def _():
                accs = [zero] * (2 * (DH // L))
                for u in range(L):
                    r = h * L + u
                    for k in range(DH // L):
                        v = buf[r, pl.ds(k * L, L)]
                        uw = plsc.bitcast(v, jnp.uint32)
                        accs[k] = accs[k] + plsc.bitcast(uw << 16,
                                                         jnp.float32)
                        accs[DH // L + k] = accs[DH // L + k] + plsc.bitcast(
                            uw & himask, jnp.float32)
                soff = seg0 * DR
                for k in range(DH // L):
                    plsc.addupdate(acc.at[pl.ds(soff + k * L, L)], accs[k])
                    plsc.addupdate(acc.at[pl.ds(soff + DH + k * L, L)],
                                   accs[DH // L + k])

            @pl.when(jnp.logical_not(uniform))
            def _():
                for u in range(L):
                    process_row(buf, h * L + u, segv[u])

            return carry

        lax.fori_loop(0, CHUNK // L, grp, 0)

    xcopy(0, xbuf0, sem0)

    def pair(jp, carry):
        do_chunk(jp * 2, xbuf0, sem0, xbuf1, sem1)

        @pl.when(jp * 2 + 1 < NC)
        def _():
            do_chunk(jp * 2 + 1, xbuf1, sem1, xbuf0, sem0)

        return carry

    lax.fori_loop(0, (NC + 1) // 2, pair, 0)

    pltpu.sync_copy(acc.at[pl.ds(0, ACC_W)],
                    out_hbm.at[pl.ds(wid * ACC_W, ACC_W)])


_sc_pool = functools.partial(
    pl.kernel,
    out_type=jax.ShapeDtypeStruct((NW * ACC_W,), jnp.float32),
    mesh=plsc.VectorSubcoreMesh(core_axis_name="c", subcore_axis_name="s",
                                num_cores=2, num_subcores=16),
    compiler_params=pltpu.CompilerParams(use_tc_tiling_on_sc=True,
                                         needs_layout_passes=False),
    scratch_types=[
        pltpu.VMEM((CHUNK, DH), jnp.float32),
        pltpu.VMEM((CHUNK, DH), jnp.float32),
        pltpu.VMEM((ACC_T,), jnp.float32),
        pltpu.VMEM((SEG_LEN + L,), jnp.int32),
        pltpu.SemaphoreType.DMA,
        pltpu.SemaphoreType.DMA,
    ],
)(_sc_pool_body)


def _reduce_parts(p_ref, o_ref):
    o_ref[...] = jnp.sum(p_ref[...], axis=0)[:, :D]


def kernel(x, segment_ids, batch_size, W, b):
    del batch_size
    seg = jnp.pad(segment_ids.astype(jnp.int32), (0, SEG_LEN + L),
                  constant_values=B_SEG)
    xw = pl.pallas_call(
        _scale_block,
        grid=(N_BLK,),
        in_specs=[
            pl.BlockSpec((D, R_BLK), lambda i: (0, i)),
            pl.BlockSpec((1, D), lambda i: (0, 0)),
            pl.BlockSpec(memory_space=pltpu.SMEM),
        ],
        out_specs=pl.BlockSpec((R_BLK, DH), lambda i: (i, 0)),
        out_shape=jax.ShapeDtypeStruct((N_PAD, DH), jnp.float32),
    )(x.T, W.reshape(1, D), b.reshape(1))
    parts = _sc_pool(xw, seg)
    parts = parts.reshape(NW, B_SEG, DR)
    out = pl.pallas_call(
        _reduce_parts,
        out_shape=jax.ShapeDtypeStruct((B_SEG, D), jnp.float32),
    )(parts)
    return out


# speedup vs baseline: 1.5113x; 1.1451x over previous
import functools

import jax
import jax.numpy as jnp
from jax import lax
from jax.experimental import pallas as pl
from jax.experimental.pallas import tpu as pltpu
from jax.experimental.pallas import tpu_sc as plsc

D = 300
L = 16
DR = 320
DH = DR // 2
B_SEG = 256
N_ROWS = 100000
NW = 32
R_BLK = 2048
N_BLK = (N_ROWS + R_BLK - 1) // R_BLK
N_PAD = N_BLK * R_BLK
CHUNK = 64
NC = 49
SEG_LEN = NC * CHUNK
ACC_W = B_SEG * DR
ACC_T = 82304


def _scale_block(xt_ref, wt_ref, b_ref, out_ref):
    xb = xt_ref[...]
    t = jax.lax.dot_general(wt_ref[...], xb, (((1,), (0,)), ((), ())),
                            preferred_element_type=jnp.float32)
    w = jax.nn.sigmoid(t + b_ref[0])
    s = jnp.transpose(xb * w, (1, 0))
    s = jnp.pad(s, ((0, 0), (0, DR - D)))
    be = jax.lax.bitcast_convert_type(s[:, :DH], jnp.uint32)
    bo = jax.lax.bitcast_convert_type(s[:, DH:], jnp.uint32)
    re = (be + 0x7FFF + ((be >> 16) & 1)) >> 16
    ro = (bo + 0x7FFF + ((bo >> 16) & 1)) & jnp.uint32(0xFFFF0000)
    out_ref[...] = jax.lax.bitcast_convert_type(re | ro, jnp.float32)


def _sc_pool_body(xw_hbm, seg_hbm, out_hbm, xbuf0, xbuf1, acc, segbuf,
                  sem0, sem1):
    cid = lax.axis_index("c")
    sid = lax.axis_index("s")
    wid = sid * 2 + cid
    rbase = ((wid * (N_ROWS // NW) + 7) // 8) * 8
    rend = (((wid + 1) * (N_ROWS // NW) + 7) // 8) * 8
    nrows = jnp.where(wid == NW - 1, N_ROWS - rbase, rend - rbase)

    pltpu.sync_copy(seg_hbm.at[pl.ds(rbase, SEG_LEN)],
                    segbuf.at[pl.ds(0, SEG_LEN)])

    zero = jnp.zeros((L,), jnp.float32)

    def zrow(i, carry):
        for u in range(8):
            acc[pl.ds((i * 8 + u) * L, L)] = zero
        return carry

    lax.fori_loop(0, ACC_T // (L * 8), zrow, 0)

    iota = lax.iota(jnp.int32, L)
    himask = jnp.full((L,), 0xFFFF0000, dtype=jnp.uint32)

    def xcopy(g, buf, sem):
        return pltpu.async_copy(xw_hbm.at[pl.ds(rbase + g * CHUNK, CHUNK), :],
                                buf, sem)

    def xwait(buf, sem):
        pltpu.make_async_copy(xw_hbm.at[pl.ds(0, CHUNK), :], buf, sem).wait()

    def process_row(buf, r, seg_scalar):
        soff = seg_scalar * DR
        for k in range(DH // L):
            v = buf[r, pl.ds(k * L, L)]
            u = plsc.bitcast(v, jnp.uint32)
            lo = plsc.bitcast(u << 16, jnp.float32)
            hi = plsc.bitcast(u & himask, jnp.float32)
            plsc.addupdate(acc.at[pl.ds(soff + k * L, L)], lo)
            plsc.addupdate(acc.at[pl.ds(soff + DH + k * L, L)], hi)

    def do_chunk(j, buf, sem, obuf, osem):
        xwait(buf, sem)

        @pl.when(j + 1 < NC)
        def _():
            xcopy(j + 1, obuf, osem)

        def grp(h, carry):
            lbase = j * CHUNK + h * L
            segv = segbuf[pl.ds(lbase, L)]
            segv = jnp.where(lbase + iota < nrows, segv, B_SEG)
            seg0 = segv[0]
            uniform = jnp.all(segv == seg0)

            @pl.when(uniform)
            def _():
                accs = [zero] * (2 * (DH // L))
                for u in range(L):
                    r = h * L + u
                    for k in range(DH // L):
                        v = buf[r, pl.ds(k * L, L)]
                        uw = plsc.bitcast(v, jnp.uint32)
                        accs[k] = accs[k] + plsc.bitcast(uw << 16,
                                                         jnp.float32)
                        accs[DH // L + k] = accs[DH // L + k] + plsc.bitcast(
                            uw & himask, jnp.float32)
                soff = seg0 * DR
                for k in range(DH // L):
                    plsc.addupdate(acc.at[pl.ds(soff + k * L, L)], accs[k])
                    plsc.addupdate(acc.at[pl.ds(soff + DH + k * L, L)],
                                   accs[DH // L + k])

            @pl.when(jnp.logical_not(uniform))
            def _():
                for u in range(L):
                    process_row(buf, h * L + u, segv[u])

            return carry

        lax.fori_loop(0, CHUNK // L, grp, 0)

    xcopy(0, xbuf0, sem0)

    def pair(jp, carry):
        do_chunk(jp * 2, xbuf0, sem0, xbuf1, sem1)

        @pl.when(jp * 2 + 1 < NC)
        def _():
            do_chunk(jp * 2 + 1, xbuf1, sem1, xbuf0, sem0)

        return carry

    lax.fori_loop(0, (NC + 1) // 2, pair, 0)

    pltpu.sync_copy(acc.at[pl.ds(0, ACC_W)],
                    out_hbm.at[pl.ds(wid * ACC_W, ACC_W)])


_sc_pool = functools.partial(
    pl.kernel,
    out_type=jax.ShapeDtypeStruct((NW * ACC_W,), jnp.float32),
    mesh=plsc.VectorSubcoreMesh(core_axis_name="c", subcore_axis_name="s",
                                num_cores=2, num_subcores=16),
    compiler_params=pltpu.CompilerParams(use_tc_tiling_on_sc=True,
                                         needs_layout_passes=False),
    scratch_types=[
        pltpu.VMEM((CHUNK, DH), jnp.float32),
        pltpu.VMEM((CHUNK, DH), jnp.float32),
        pltpu.VMEM((ACC_T,), jnp.float32),
        pltpu.VMEM((SEG_LEN + L,), jnp.int32),
        pltpu.SemaphoreType.DMA,
        pltpu.SemaphoreType.DMA,
    ],
)(_sc_pool_body)


def _reduce_parts(p_ref, o_ref):
    o_ref[...] = jnp.sum(p_ref[...], axis=0)[:, :D]


def kernel(x, segment_ids, batch_size, W, b):
    del batch_size
    seg = jnp.pad(segment_ids.astype(jnp.int32), (0, SEG_LEN + L),
                  constant_values=B_SEG)
    xw = pl.pallas_call(
        _scale_block,
        grid=(N_BLK,),
        in_specs=[
            pl.BlockSpec((D, R_BLK), lambda i: (0, i)),
            pl.BlockSpec((1, D), lambda i: (0, 0)),
            pl.BlockSpec(memory_space=pltpu.SMEM),
        ],
        out_specs=pl.BlockSpec((R_BLK, DH), lambda i: (i, 0)),
        out_shape=jax.ShapeDtypeStruct((N_PAD, DH), jnp.float32),
    )(x.T, W.reshape(1, D), b.reshape(1))
    parts = _sc_pool(xw, seg)
    parts = parts.reshape(NW, B_SEG, DR)
    out = pl.pallas_call(
        _reduce_parts,
        out_shape=jax.ShapeDtypeStruct((B_SEG, D), jnp.float32),
    )(parts)
    return out


# speedup vs baseline: 1.6150x; 1.0686x over previous
import functools

import jax
import jax.numpy as jnp
from jax import lax
from jax.experimental import pallas as pl
from jax.experimental.pallas import tpu as pltpu
from jax.experimental.pallas import tpu_sc as plsc

D = 300
L = 16
DR = 320
DH = DR // 2
B_SEG = 256
N_ROWS = 100000
NW = 32
R_BLK = 4096
N_BLK = (N_ROWS + R_BLK - 1) // R_BLK
N_PAD = N_BLK * R_BLK
CHUNK = 64
NC = 49
SEG_LEN = NC * CHUNK
ACC_W = B_SEG * DR
ACC_T = 82304


def _scale_block(xt_ref, wt_ref, b_ref, out_ref):
    xb = xt_ref[...]
    t = jax.lax.dot_general(wt_ref[...], xb, (((1,), (0,)), ((), ())),
                            preferred_element_type=jnp.float32)
    w = jax.nn.sigmoid(t + b_ref[0])
    s = jnp.transpose(xb * w, (1, 0))
    s = jnp.pad(s, ((0, 0), (0, DR - D)))
    be = jax.lax.bitcast_convert_type(s[:, :DH], jnp.uint32)
    bo = jax.lax.bitcast_convert_type(s[:, DH:], jnp.uint32)
    re = (be + 0x7FFF + ((be >> 16) & 1)) >> 16
    ro = (bo + 0x7FFF + ((bo >> 16) & 1)) & jnp.uint32(0xFFFF0000)
    out_ref[...] = jax.lax.bitcast_convert_type(re | ro, jnp.float32)


def _sc_pool_body(xw_hbm, seg_hbm, out_hbm, xbuf0, xbuf1, acc, segbuf,
                  sem0, sem1):
    cid = lax.axis_index("c")
    sid = lax.axis_index("s")
    wid = sid * 2 + cid
    rbase = ((wid * (N_ROWS // NW) + 7) // 8) * 8
    rend = (((wid + 1) * (N_ROWS // NW) + 7) // 8) * 8
    nrows = jnp.where(wid == NW - 1, N_ROWS - rbase, rend - rbase)

    pltpu.sync_copy(seg_hbm.at[pl.ds(rbase, SEG_LEN)],
                    segbuf.at[pl.ds(0, SEG_LEN)])

    zero = jnp.zeros((L,), jnp.float32)

    def zrow(i, carry):
        for u in range(8):
            acc[pl.ds((i * 8 + u) * L, L)] = zero
        return carry

    lax.fori_loop(0, ACC_T // (L * 8), zrow, 0)

    iota = lax.iota(jnp.int32, L)
    himask = jnp.full((L,), 0xFFFF0000, dtype=jnp.uint32)

    def xcopy(g, buf, sem):
        return pltpu.async_copy(xw_hbm.at[pl.ds(rbase + g * CHUNK, CHUNK), :],
                                buf, sem)

    def xwait(buf, sem):
        pltpu.make_async_copy(xw_hbm.at[pl.ds(0, CHUNK), :], buf, sem).wait()

    def process_row(buf, r, seg_scalar):
        soff = seg_scalar * DR
        for k in range(DH // L):
            v = buf[r, pl.ds(k * L, L)]
            u = plsc.bitcast(v, jnp.uint32)
            lo = plsc.bitcast(u << 16, jnp.float32)
            hi = plsc.bitcast(u & himask, jnp.float32)
            plsc.addupdate(acc.at[pl.ds(soff + k * L, L)], lo)
            plsc.addupdate(acc.at[pl.ds(soff + DH + k * L, L)], hi)

    def do_chunk(j, buf, sem, obuf, osem):
        xwait(buf, sem)

        @pl.when(j + 1 < NC)
        def _():
            xcopy(j + 1, obuf, osem)

        def grp(h, carry):
            lbase = j * CHUNK + h * L
            segv = segbuf[pl.ds(lbase, L)]
            segv = jnp.where(lbase + iota < nrows, segv, B_SEG)
            seg0 = segv[0]
            uniform = jnp.all(segv == seg0)

            @pl.when(uniform)
            def _():
                accs = [zero] * (2 * (DH // L))
                for u in range(L):
                    r = h * L + u
                    for k in range(DH // L):
                        v = buf[r, pl.ds(k * L, L)]
                        uw = plsc.bitcast(v, jnp.uint32)
                        accs[k] = accs[k] + plsc.bitcast(uw << 16,
                                                         jnp.float32)
                        accs[DH // L + k] = accs[DH // L + k] + plsc.bitcast(
                            uw & himask, jnp.float32)
                soff = seg0 * DR
                for k in range(DH // L):
                    plsc.addupdate(acc.at[pl.ds(soff + k * L, L)], accs[k])
                    plsc.addupdate(acc.at[pl.ds(soff + DH + k * L, L)],
                                   accs[DH // L + k])

            @pl.when(jnp.logical_not(uniform))
            def _():
                for u in range(L):
                    process_row(buf, h * L + u, segv[u])

            return carry

        lax.fori_loop(0, CHUNK // L, grp, 0)

    xcopy(0, xbuf0, sem0)

    def pair(jp, carry):
        do_chunk(jp * 2, xbuf0, sem0, xbuf1, sem1)

        @pl.when(jp * 2 + 1 < NC)
        def _():
            do_chunk(jp * 2 + 1, xbuf1, sem1, xbuf0, sem0)

        return carry

    lax.fori_loop(0, (NC + 1) // 2, pair, 0)

    pltpu.sync_copy(acc.at[pl.ds(0, ACC_W)],
                    out_hbm.at[pl.ds(wid * ACC_W, ACC_W)])


_sc_pool = functools.partial(
    pl.kernel,
    out_type=jax.ShapeDtypeStruct((NW * ACC_W,), jnp.float32),
    mesh=plsc.VectorSubcoreMesh(core_axis_name="c", subcore_axis_name="s",
                                num_cores=2, num_subcores=16),
    compiler_params=pltpu.CompilerParams(use_tc_tiling_on_sc=True,
                                         needs_layout_passes=False),
    scratch_types=[
        pltpu.VMEM((CHUNK, DH), jnp.float32),
        pltpu.VMEM((CHUNK, DH), jnp.float32),
        pltpu.VMEM((ACC_T,), jnp.float32),
        pltpu.VMEM((SEG_LEN + L,), jnp.int32),
        pltpu.SemaphoreType.DMA,
        pltpu.SemaphoreType.DMA,
    ],
)(_sc_pool_body)


def _reduce_parts(p_ref, o_ref):
    o_ref[...] = jnp.sum(p_ref[...], axis=0)[:, :D]


def kernel(x, segment_ids, batch_size, W, b):
    del batch_size
    seg = jnp.pad(segment_ids.astype(jnp.int32), (0, SEG_LEN + L),
                  constant_values=B_SEG)
    xw = pl.pallas_call(
        _scale_block,
        grid=(N_BLK,),
        in_specs=[
            pl.BlockSpec((D, R_BLK), lambda i: (0, i)),
            pl.BlockSpec((1, D), lambda i: (0, 0)),
            pl.BlockSpec(memory_space=pltpu.SMEM),
        ],
        out_specs=pl.BlockSpec((R_BLK, DH), lambda i: (i, 0)),
        out_shape=jax.ShapeDtypeStruct((N_PAD, DH), jnp.float32),
    )(x.T, W.reshape(1, D), b.reshape(1))
    parts = _sc_pool(xw, seg)
    parts = parts.reshape(NW, B_SEG, DR)
    out = pl.pallas_call(
        _reduce_parts,
        out_shape=jax.ShapeDtypeStruct((B_SEG, D), jnp.float32),
    )(parts)
    return out


# speedup vs baseline: 1.6504x; 1.0219x over previous
import functools

import jax
import jax.numpy as jnp
from jax import lax
from jax.experimental import pallas as pl
from jax.experimental.pallas import tpu as pltpu
from jax.experimental.pallas import tpu_sc as plsc

D = 300
L = 16
DR = 320
DH = DR // 2
B_SEG = 256
N_ROWS = 100000
NW = 32
R_BLK = 8192
N_BLK = (N_ROWS + R_BLK - 1) // R_BLK
N_PAD = N_BLK * R_BLK
CHUNK = 64
NC = 49
SEG_LEN = NC * CHUNK
ACC_W = B_SEG * DR
ACC_T = 82304


def _scale_block(xt_ref, wt_ref, b_ref, out_ref):
    xb = xt_ref[...]
    t = jax.lax.dot_general(wt_ref[...], xb, (((1,), (0,)), ((), ())),
                            preferred_element_type=jnp.float32)
    w = jax.nn.sigmoid(t + b_ref[0])
    s = jnp.transpose(xb * w, (1, 0))
    s = jnp.pad(s, ((0, 0), (0, DR - D)))
    be = jax.lax.bitcast_convert_type(s[:, :DH], jnp.uint32)
    bo = jax.lax.bitcast_convert_type(s[:, DH:], jnp.uint32)
    re = (be + 0x7FFF + ((be >> 16) & 1)) >> 16
    ro = (bo + 0x7FFF + ((bo >> 16) & 1)) & jnp.uint32(0xFFFF0000)
    out_ref[...] = jax.lax.bitcast_convert_type(re | ro, jnp.float32)


def _sc_pool_body(xw_hbm, seg_hbm, out_hbm, xbuf0, xbuf1, acc, segbuf,
                  sem0, sem1):
    cid = lax.axis_index("c")
    sid = lax.axis_index("s")
    wid = sid * 2 + cid
    rbase = ((wid * (N_ROWS // NW) + 7) // 8) * 8
    rend = (((wid + 1) * (N_ROWS // NW) + 7) // 8) * 8
    nrows = jnp.where(wid == NW - 1, N_ROWS - rbase, rend - rbase)

    pltpu.sync_copy(seg_hbm.at[pl.ds(rbase, SEG_LEN)],
                    segbuf.at[pl.ds(0, SEG_LEN)])

    zero = jnp.zeros((L,), jnp.float32)

    def zrow(i, carry):
        for u in range(8):
            acc[pl.ds((i * 8 + u) * L, L)] = zero
        return carry

    lax.fori_loop(0, ACC_T // (L * 8), zrow, 0)

    iota = lax.iota(jnp.int32, L)
    himask = jnp.full((L,), 0xFFFF0000, dtype=jnp.uint32)

    def xcopy(g, buf, sem):
        return pltpu.async_copy(xw_hbm.at[pl.ds(rbase + g * CHUNK, CHUNK), :],
                                buf, sem)

    def xwait(buf, sem):
        pltpu.make_async_copy(xw_hbm.at[pl.ds(0, CHUNK), :], buf, sem).wait()

    def process_row(buf, r, seg_scalar):
        soff = seg_scalar * DR
        for k in range(DH // L):
            v = buf[r, pl.ds(k * L, L)]
            u = plsc.bitcast(v, jnp.uint32)
            lo = plsc.bitcast(u << 16, jnp.float32)
            hi = plsc.bitcast(u & himask, jnp.float32)
            plsc.addupdate(acc.at[pl.ds(soff + k * L, L)], lo)
            plsc.addupdate(acc.at[pl.ds(soff + DH + k * L, L)], hi)

    def do_chunk(j, buf, sem, obuf, osem):
        xwait(buf, sem)

        @pl.when(j + 1 < NC)
        def _():
            xcopy(j + 1, obuf, osem)

        def grp(h, carry):
            lbase = j * CHUNK + h * L
            segv = segbuf[pl.ds(lbase, L)]
            segv = jnp.where(lbase + iota < nrows, segv, B_SEG)
            seg0 = segv[0]
            uniform = jnp.all(segv == seg0)

            @pl.when(uniform)
            def _():
                accs = [zero] * (2 * (DH // L))
                for u in range(L):
                    r = h * L + u
                    for k in range(DH // L):
                        v = buf[r, pl.ds(k * L, L)]
                        uw = plsc.bitcast(v, jnp.uint32)
                        accs[k] = accs[k] + plsc.bitcast(uw << 16,
                                                         jnp.float32)
                        accs[DH // L + k] = accs[DH // L + k] + plsc.bitcast(
                            uw & himask, jnp.float32)
                soff = seg0 * DR
                for k in range(DH // L):
                    plsc.addupdate(acc.at[pl.ds(soff + k * L, L)], accs[k])
                    plsc.addupdate(acc.at[pl.ds(soff + DH + k * L, L)],
                                   accs[DH // L + k])

            @pl.when(jnp.logical_not(uniform))
            def _():
                for u in range(L):
                    process_row(buf, h * L + u, segv[u])

            return carry

        lax.fori_loop(0, CHUNK // L, grp, 0)

    xcopy(0, xbuf0, sem0)

    def pair(jp, carry):
        do_chunk(jp * 2, xbuf0, sem0, xbuf1, sem1)

        @pl.when(jp * 2 + 1 < NC)
        def _():
            do_chunk(jp * 2 + 1, xbuf1, sem1, xbuf0, sem0)

        return carry

    lax.fori_loop(0, (NC + 1) // 2, pair, 0)

    pltpu.sync_copy(acc.at[pl.ds(0, ACC_W)],
                    out_hbm.at[pl.ds(wid * ACC_W, ACC_W)])


_sc_pool = functools.partial(
    pl.kernel,
    out_type=jax.ShapeDtypeStruct((NW * ACC_W,), jnp.float32),
    mesh=plsc.VectorSubcoreMesh(core_axis_name="c", subcore_axis_name="s",
                                num_cores=2, num_subcores=16),
    compiler_params=pltpu.CompilerParams(use_tc_tiling_on_sc=True,
                                         needs_layout_passes=False),
    scratch_types=[
        pltpu.VMEM((CHUNK, DH), jnp.float32),
        pltpu.VMEM((CHUNK, DH), jnp.float32),
        pltpu.VMEM((ACC_T,), jnp.float32),
        pltpu.VMEM((SEG_LEN + L,), jnp.int32),
        pltpu.SemaphoreType.DMA,
        pltpu.SemaphoreType.DMA,
    ],
)(_sc_pool_body)


def _reduce_parts(p_ref, o_ref):
    o_ref[...] = jnp.sum(p_ref[...], axis=0)[:, :D]


def kernel(x, segment_ids, batch_size, W, b):
    del batch_size
    seg = jnp.pad(segment_ids.astype(jnp.int32), (0, SEG_LEN + L),
                  constant_values=B_SEG)
    xw = pl.pallas_call(
        _scale_block,
        grid=(N_BLK,),
        in_specs=[
            pl.BlockSpec((D, R_BLK), lambda i: (0, i)),
            pl.BlockSpec((1, D), lambda i: (0, 0)),
            pl.BlockSpec(memory_space=pltpu.SMEM),
        ],
        out_specs=pl.BlockSpec((R_BLK, DH), lambda i: (i, 0)),
        out_shape=jax.ShapeDtypeStruct((N_PAD, DH), jnp.float32),
    )(x.T, W.reshape(1, D), b.reshape(1))
    parts = _sc_pool(xw, seg)
    parts = parts.reshape(NW, B_SEG, DR)
    out = pl.pallas_call(
        _reduce_parts,
        out_shape=jax.ShapeDtypeStruct((B_SEG, D), jnp.float32),
    )(parts)
    return out
